# baseline (speedup 1.0000x reference)
import jax
import jax.numpy as jnp
from jax import lax
from jax.experimental import pallas as pl
from jax.experimental.pallas import tpu as pltpu

C = 512
C2 = 256
HW = 4096
QB = 256
NQ = HW // QB


def _body(x2q_ref, x2p_ref, x2f_ref, maskc_ref, maskq_ref, out_ref, lnorm_ref,
          cos_ref):
    i = pl.program_id(0)

    @pl.when(i == 0)
    def _():
        latf = x2f_ref[C2:, :]
        n2 = jnp.sum(latf * latf, axis=0, keepdims=True)
        norm = jnp.sqrt(n2) + 1e-8
        lnorm_ref[...] = latf / norm

    @pl.when(i < NQ)
    def _():
        latq = x2q_ref[C2:, :]
        cos_ref[pl.ds((i % 2) * HW, HW), :] = lax.dot_general(
            lnorm_ref[...], latq,
            (((0,), (0,)), ((), ())),
            preferred_element_type=jnp.float32,
        )

    @pl.when(i > 0)
    def _():
        j = (i + 1) % 2
        cosT = cos_ref[pl.ds(j * HW, HW), :]
        cosm = jnp.where(maskc_ref[...] != 0, -jnp.inf, cosT)
        m = jnp.max(cosm, axis=0, keepdims=True)
        cio = lax.broadcasted_iota(jnp.int32, (HW, QB), 0)
        idx = jnp.min(
            jnp.where(cosm == m, cio, jnp.int32(HW)), axis=0, keepdims=True
        )
        oh = ((cio == idx) & (maskq_ref[...] != 0)).astype(jnp.float32)
        shift = lax.dot_general(
            x2f_ref[:C2, :], oh,
            (((1,), (0,)), ((), ())),
            preferred_element_type=jnp.float32,
        )
        out_ref[:C, :] = x2p_ref[...]
        out_ref[C:, :] = shift


def kernel(input, mask):
    b, c, h, w = input.shape
    x2 = input.reshape(C, HW)
    mask_col = mask.reshape(HW, 1)
    mask_row = mask.reshape(1, HW)

    def qmap(i):
        return (0, jnp.minimum(i, NQ - 1))

    def cmap(i):
        return (0, jnp.maximum(i - 1, 0))

    out = pl.pallas_call(
        _body,
        grid=(NQ + 1,),
        in_specs=[
            pl.BlockSpec((C, QB), qmap),
            pl.BlockSpec((C, QB), cmap),
            pl.BlockSpec((C, HW), lambda i: (0, 0)),
            pl.BlockSpec((HW, 1), lambda i: (0, 0)),
            pl.BlockSpec((1, QB), cmap),
        ],
        out_specs=pl.BlockSpec((C + C2, QB), cmap),
        out_shape=jax.ShapeDtypeStruct((C + C2, HW), jnp.float32),
        scratch_shapes=[
            pltpu.VMEM((C2, HW), jnp.float32),
            pltpu.VMEM((2 * HW, QB), jnp.float32),
        ],
    )(x2, x2, x2, mask_col, mask_row)
    return out.reshape(b, C + C2, h, w)


# speedup vs baseline: 1.1564x; 1.1564x over previous
import jax
import jax.numpy as jnp
from jax import lax
from jax.experimental import pallas as pl
from jax.experimental.pallas import tpu as pltpu

C = 512
C2 = 256
HW = 4096
QB = 512
NQ = HW // QB


def _body(x2q_ref, x2p_ref, x2f_ref, maskc_ref, maskq_ref, out_ref, lnorm_ref,
          cos_ref):
    i = pl.program_id(0)

    @pl.when(i == 0)
    def _():
        latf = x2f_ref[C2:, :]
        n2 = jnp.sum(latf * latf, axis=0, keepdims=True)
        norm = jnp.sqrt(n2) + 1e-8
        lnorm_ref[...] = latf / norm

    @pl.when(i < NQ)
    def _():
        latq = x2q_ref[C2:, :]
        cos_ref[pl.ds((i % 2) * HW, HW), :] = lax.dot_general(
            lnorm_ref[...], latq,
            (((0,), (0,)), ((), ())),
            preferred_element_type=jnp.float32,
        )

    @pl.when(i > 0)
    def _():
        j = (i + 1) % 2
        cosT = cos_ref[pl.ds(j * HW, HW), :]
        cosm = jnp.where(maskc_ref[...] != 0, -jnp.inf, cosT)
        m = jnp.max(cosm, axis=0, keepdims=True)
        cio = lax.broadcasted_iota(jnp.int32, (HW, QB), 0)
        idx = jnp.min(
            jnp.where(cosm == m, cio, jnp.int32(HW)), axis=0, keepdims=True
        )
        oh = ((cio == idx) & (maskq_ref[...] != 0)).astype(jnp.float32)
        shift = lax.dot_general(
            x2f_ref[:C2, :], oh,
            (((1,), (0,)), ((), ())),
            preferred_element_type=jnp.float32,
        )
        out_ref[:C, :] = x2p_ref[...]
        out_ref[C:, :] = shift


def kernel(input, mask):
    b, c, h, w = input.shape
    x2 = input.reshape(C, HW)
    mask_col = mask.reshape(HW, 1)
    mask_row = mask.reshape(1, HW)

    def qmap(i):
        return (0, jnp.minimum(i, NQ - 1))

    def cmap(i):
        return (0, jnp.maximum(i - 1, 0))

    out = pl.pallas_call(
        _body,
        grid=(NQ + 1,),
        in_specs=[
            pl.BlockSpec((C, QB), qmap),
            pl.BlockSpec((C, QB), cmap),
            pl.BlockSpec((C, HW), lambda i: (0, 0)),
            pl.BlockSpec((HW, 1), lambda i: (0, 0)),
            pl.BlockSpec((1, QB), cmap),
        ],
        out_specs=pl.BlockSpec((C + C2, QB), cmap),
        out_shape=jax.ShapeDtypeStruct((C + C2, HW), jnp.float32),
        scratch_shapes=[
            pltpu.VMEM((C2, HW), jnp.float32),
            pltpu.VMEM((2 * HW, QB), jnp.float32),
        ],
    )(x2, x2, x2, mask_col, mask_row)
    return out.reshape(b, C + C2, h, w)
